# baseline (speedup 1.0000x reference)
import functools

import jax
import jax.numpy as jnp
from jax import lax
from jax.experimental import pallas as pl
from jax.experimental.pallas import tpu as pltpu
from jax.experimental.pallas import tpu_sc as plsc

N = 10000
E = 320000
D = 128
DE = 16
L = 3
G = 64
OUT = 16
BN_EPS = 1e-5

NC = 2
NS = 16
NW = NC * NS
EPW = E // NW
CH = 64
NFULL = EPW // CH
NPAIR = NFULL // 2
TAIL = EPW - NFULL * CH


_EB = 4000
DW = D // 2

_EPERM = (
    [32 * (q // 16) + q % 16 for q in range(DW)]
    + [32 * (q // 16) + 16 + q % 16 for q in range(DW)]
)


def _edge_proj_body(*refs):
    ea_ref, wlo_ref, whi_ref, blo_ref, bhi_ref = refs[:5]
    outs = refs[5:]
    ea = ea_ref[...]
    half = jnp.int32(0x8000)
    for i, out in enumerate(outs):
        eflo = (
            jnp.dot(ea, wlo_ref[i], preferred_element_type=jnp.float32)
            + blo_ref[i]
        )
        efhi = (
            jnp.dot(ea, whi_ref[i], preferred_element_type=jnp.float32)
            + bhi_ref[i]
        )
        ulo = lax.bitcast_convert_type(eflo, jnp.int32) + half
        uhi = lax.bitcast_convert_type(efhi, jnp.int32) + half
        lo = (ulo >> 16) & jnp.int32(0xFFFF)
        hi = uhi & jnp.int32(-65536)
        out[...] = hi | lo


def _edge_proj(edge_attr, We_lo, We_hi, be_lo, be_hi):
    nl = We_lo.shape[0]
    grid = (E // _EB,)
    return pl.pallas_call(
        _edge_proj_body,
        grid=grid,
        in_specs=[
            pl.BlockSpec((_EB, DE), lambda i: (i, 0)),
            pl.BlockSpec((nl, DE, DW), lambda i: (0, 0, 0)),
            pl.BlockSpec((nl, DE, DW), lambda i: (0, 0, 0)),
            pl.BlockSpec((nl, DW), lambda i: (0, 0)),
            pl.BlockSpec((nl, DW), lambda i: (0, 0)),
        ],
        out_specs=[pl.BlockSpec((_EB, DW), lambda i: (i, 0))] * nl,
        out_shape=[jax.ShapeDtypeStruct((E, DW), jnp.int32)] * nl,
    )(edge_attr, We_lo, We_hi, be_lo, be_hi)


def _edge_row_update(mrow, erow, j):
    for g in range(D // 32):
        w = erow[j, pl.ds(16 * g, 16)]
        lo = lax.bitcast_convert_type(w << 16, jnp.float32)
        hi = lax.bitcast_convert_type(w & jnp.int32(-65536), jnp.float32)
        sl_lo = pl.ds(32 * g, 16)
        sl_hi = pl.ds(32 * g + 16, 16)
        mrow[j, sl_lo] = jnp.maximum(mrow[j, sl_lo] + lo, 0.0)
        mrow[j, sl_hi] = jnp.maximum(mrow[j, sl_hi] + hi, 0.0)


@functools.partial(
    pl.kernel,
    out_type=jax.ShapeDtypeStruct((NC, N, D), jnp.float32),
    mesh=plsc.VectorSubcoreMesh(
        core_axis_name="c", subcore_axis_name="s", num_cores=NC, num_subcores=NS
    ),
    scratch_types=[
        [pltpu.VMEM((CH,), jnp.int32)] * 2,
        [pltpu.VMEM((CH,), jnp.int32)] * 2,
        [pltpu.VMEM((CH,), jnp.int32)] * 2,
        [pltpu.VMEM((CH, DW), jnp.int32)] * 2,
        [pltpu.VMEM((CH, D), jnp.float32)] * 2,
        pltpu.VMEM((TAIL,), jnp.int32),
        pltpu.VMEM((TAIL,), jnp.int32),
        pltpu.VMEM((TAIL, DW), jnp.int32),
        pltpu.VMEM((TAIL, D), jnp.float32),
        pltpu.VMEM_SHARED((N, D), jnp.float32),
        [pltpu.SemaphoreType.DMA] * 2,
        [pltpu.SemaphoreType.DMA] * 2,
        [pltpu.SemaphoreType.DMA] * 2,
        [pltpu.SemaphoreType.DMA] * 2,
        pltpu.SemaphoreType.DMA,
    ],
)
def _sc_edge_pass(h_hbm, e_hbm, src_hbm, dst_hbm, out_hbm,
                  sidx, didx, dscat, ebuf, mbuf,
                  tsidx, tdidx, tebuf, tmbuf,
                  aggr, i_sem, e_sem, g_sem, s_sem, t_sem):
    c = lax.axis_index("c")
    s = lax.axis_index("s")
    wid = c * NS + s

    def chunk_base(t):
        return pl.multiple_of(wid * EPW + t * CH, 8)

    def issue_idx_e(t, b):
        base = chunk_base(t)
        pltpu.async_copy(src_hbm.at[pl.ds(base, CH)], sidx[b], i_sem[b])
        pltpu.async_copy(dst_hbm.at[pl.ds(base, CH)], didx[b], i_sem[b])
        pltpu.async_copy(e_hbm.at[pl.ds(base, CH), :], ebuf[b], e_sem[b])

    def wait_idx(t, b):
        base = chunk_base(t)
        pltpu.make_async_copy(
            src_hbm.at[pl.ds(base, CH)], sidx[b], i_sem[b]).wait()
        pltpu.make_async_copy(
            dst_hbm.at[pl.ds(base, CH)], didx[b], i_sem[b]).wait()

    def drain_scatter(b):
        pltpu.make_async_copy(mbuf[b], aggr.at[dscat[b]], s_sem[b]).wait()

    issue_idx_e(0, 0)
    issue_idx_e(1, 1)

    @pl.when(s == 0)
    def _init():
        pltpu.sync_copy(h_hbm, aggr)

    plsc.subcore_barrier()

    wait_idx(0, 0)
    pltpu.async_copy(h_hbm.at[sidx[0]], mbuf[0], g_sem[0])

    def pair_body(i, carry):
        for b in range(2):
            t = 2 * i + b
            base = chunk_base(t)
            nb = 1 - b

            def _launch_next():
                wait_idx(t + 1, nb)
                if b == 0:
                    @pl.when(i >= 1)
                    def _drain_prev():
                        drain_scatter(nb)
                else:
                    drain_scatter(nb)
                pltpu.async_copy(h_hbm.at[sidx[nb]], mbuf[nb], g_sem[nb])

            if b == 0:
                _launch_next()
            else:
                pl.when(i < NPAIR - 1)(_launch_next)

            pltpu.make_async_copy(
                e_hbm.at[pl.ds(base, CH), :], ebuf[b], e_sem[b]).wait()
            pltpu.make_async_copy(
                h_hbm.at[sidx[b]], mbuf[b], g_sem[b]).wait()

            @plsc.parallel_loop(0, CH, 1, unroll=4)
            def _compute(j):
                _edge_row_update(mbuf[b], ebuf[b], j)

            for k in range(CH // 16):
                sl = pl.ds(k * 16, 16)
                dscat[b][sl] = didx[b][sl]
            pltpu.async_copy(mbuf[b], aggr.at[dscat[b]], s_sem[b], add=True)

            @pl.when(i < NPAIR - 1)
            def _prefetch():
                issue_idx_e(t + 2, b)
        return carry

    lax.fori_loop(0, NPAIR, pair_body, 0)

    for b in range(2):
        drain_scatter(b)

    tbase = pl.multiple_of(wid * EPW + NFULL * CH, 8)
    pltpu.sync_copy(src_hbm.at[pl.ds(tbase, TAIL)], tsidx)
    pltpu.sync_copy(dst_hbm.at[pl.ds(tbase, TAIL)], tdidx)
    pltpu.sync_copy(e_hbm.at[pl.ds(tbase, TAIL), :], tebuf)
    pltpu.async_copy(h_hbm.at[tsidx], tmbuf, t_sem).wait()

    def tail_body(j, carry):
        _edge_row_update(tmbuf, tebuf, j)
        return carry

    lax.fori_loop(0, TAIL, tail_body, 0)
    pltpu.sync_copy(tmbuf, aggr.at[tdidx], add=True)

    plsc.subcore_barrier()

    rows = 624

    @pl.when(s < NS - 1)
    def _copy_main():
        off = pl.multiple_of(s * rows, 8)
        pltpu.sync_copy(
            aggr.at[pl.ds(off, rows), :],
            out_hbm.at[c, pl.ds(off, rows), :],
        )

    @pl.when(s == NS - 1)
    def _copy_tail():
        off = (NS - 1) * rows
        pltpu.sync_copy(
            aggr.at[pl.ds(off, N - off), :],
            out_hbm.at[c, pl.ds(off, N - off), :],
        )


_NB = 1000


def _dense_body(agg_ref, h_ref, w1_ref, b1_ref, w2_ref, b2_ref,
                sc_ref, sh_ref, out_ref):
    x = agg_ref[0] + agg_ref[1] - h_ref[...]
    x = jnp.maximum(
        jnp.dot(x, w1_ref[...], preferred_element_type=jnp.float32) + b1_ref[...],
        0.0,
    )
    x = jnp.maximum(
        jnp.dot(x, w2_ref[...], preferred_element_type=jnp.float32) + b2_ref[...],
        0.0,
    )
    x = x * sc_ref[...] + sh_ref[...]
    out_ref[...] = jnp.maximum(x, 0.0)


def _dense(agg, h, w1, b1, w2, b2, scale, shift):
    grid = (N // _NB,)
    return pl.pallas_call(
        _dense_body,
        grid=grid,
        in_specs=[
            pl.BlockSpec((NC, _NB, D), lambda i: (0, i, 0)),
            pl.BlockSpec((_NB, D), lambda i: (i, 0)),
            pl.BlockSpec((D, D), lambda i: (0, 0)),
            pl.BlockSpec((D,), lambda i: (0,)),
            pl.BlockSpec((D, D), lambda i: (0, 0)),
            pl.BlockSpec((D,), lambda i: (0,)),
            pl.BlockSpec((D,), lambda i: (0,)),
            pl.BlockSpec((D,), lambda i: (0,)),
        ],
        out_specs=pl.BlockSpec((_NB, D), lambda i: (i, 0)),
        out_shape=jax.ShapeDtypeStruct((N, D), jnp.float32),
    )(agg, h, w1, b1, w2, b2, scale, shift)


def _pool_body(h_ref, batch_ref, wc1_ref, bc1_ref, wc2_ref, bc2_ref,
               out_ref, sums_ref, cnts_ref):
    i = pl.program_id(0)

    @pl.when(i == 0)
    def _init():
        sums_ref[...] = jnp.zeros_like(sums_ref)
        cnts_ref[...] = jnp.zeros_like(cnts_ref)

    b = batch_ref[...]
    gids = lax.broadcasted_iota(jnp.int32, (_NB, G), 1)
    onehot = (gids == b).astype(jnp.float32)
    dn = (((0,), (0,)), ((), ()))
    sums_ref[...] += lax.dot_general(
        onehot, h_ref[...], dn, preferred_element_type=jnp.float32
    )
    cnts_ref[...] += lax.dot_general(
        onehot, jnp.ones_like(h_ref), dn, preferred_element_type=jnp.float32
    )

    @pl.when(i == (N // _NB) - 1)
    def _finish():
        pooled = sums_ref[...] / jnp.maximum(cnts_ref[...], 1.0)
        hid = jnp.maximum(
            jnp.dot(pooled, wc1_ref[...], preferred_element_type=jnp.float32)
            + bc1_ref[...],
            0.0,
        )
        out_ref[...] = (
            jnp.dot(hid, wc2_ref[...], preferred_element_type=jnp.float32)
            + bc2_ref[...]
        )


def _pool_classify(h, batch2d, wc1, bc1, wc2, bc2):
    grid = (N // _NB,)
    return pl.pallas_call(
        _pool_body,
        grid=grid,
        in_specs=[
            pl.BlockSpec((_NB, D), lambda i: (i, 0)),
            pl.BlockSpec((_NB, 1), lambda i: (i, 0)),
            pl.BlockSpec((D, D), lambda i: (0, 0)),
            pl.BlockSpec((D,), lambda i: (0,)),
            pl.BlockSpec((D, OUT), lambda i: (0, 0)),
            pl.BlockSpec((OUT,), lambda i: (0,)),
        ],
        out_specs=pl.BlockSpec((G, OUT), lambda i: (0, 0)),
        out_shape=jax.ShapeDtypeStruct((G, OUT), jnp.float32),
        scratch_shapes=[
            pltpu.VMEM((G, D), jnp.float32),
            pltpu.VMEM((G, D), jnp.float32),
        ],
    )(h, batch2d, wc1, bc1, wc2, bc2)


def kernel(x, edge_attr, W1, b1, W2, b2, We, be, bn_gamma, bn_beta,
           bn_mean, bn_var, Wc1, bc1, Wc2, bc2, edge_index, batch):
    src = edge_index[0]
    dst = edge_index[1]
    perm = jnp.asarray(_EPERM, dtype=jnp.int32)
    We_p = We[:, :, perm]
    be_p = be[:, perm]
    (e0,) = _edge_proj(edge_attr, We_p[0:1, :, :DW], We_p[0:1, :, DW:],
                       be_p[0:1, :DW], be_p[0:1, DW:])
    e1, e2 = _edge_proj(edge_attr, We_p[1:3, :, :DW], We_p[1:3, :, DW:],
                        be_p[1:3, :DW], be_p[1:3, DW:])
    es = (e0, e1, e2)
    scale = bn_gamma / jnp.sqrt(bn_var + BN_EPS)
    shift = bn_beta - bn_mean * scale

    h = x
    for i in range(L):
        agg = _sc_edge_pass(h, es[i], src, dst)
        h = _dense(agg, h, W1[i], b1[i], W2[i], b2[i], scale[i], shift[i])

    return _pool_classify(h, batch.reshape(N, 1), Wc1, bc1, Wc2, bc2)


# speedup vs baseline: 1.0368x; 1.0368x over previous
import functools

import jax
import jax.numpy as jnp
from jax import lax
from jax.experimental import pallas as pl
from jax.experimental.pallas import tpu as pltpu
from jax.experimental.pallas import tpu_sc as plsc

N = 10000
E = 320000
D = 128
DE = 16
L = 3
G = 64
OUT = 16
BN_EPS = 1e-5

NC = 2
NS = 16
NW = NC * NS
EPW = E // NW
CH = 64
NFULL = EPW // CH
NPAIR = NFULL // 2
TAIL = EPW - NFULL * CH


_EB = 4000
DW = D // 2

_EPERM = (
    [32 * (q // 16) + q % 16 for q in range(DW)]
    + [32 * (q // 16) + 16 + q % 16 for q in range(DW)]
)


def _edge_proj_body(*refs):
    ea_ref, wlo_ref, whi_ref, blo_ref, bhi_ref = refs[:5]
    outs = refs[5:]
    ea = ea_ref[...]
    half = jnp.int32(0x8000)
    for i, out in enumerate(outs):
        eflo = (
            jnp.dot(ea, wlo_ref[i], preferred_element_type=jnp.float32)
            + blo_ref[i]
        )
        efhi = (
            jnp.dot(ea, whi_ref[i], preferred_element_type=jnp.float32)
            + bhi_ref[i]
        )
        ulo = lax.bitcast_convert_type(eflo, jnp.int32) + half
        uhi = lax.bitcast_convert_type(efhi, jnp.int32) + half
        lo = (ulo >> 16) & jnp.int32(0xFFFF)
        hi = uhi & jnp.int32(-65536)
        out[...] = hi | lo


def _edge_proj(edge_attr, We_lo, We_hi, be_lo, be_hi):
    nl = We_lo.shape[0]
    grid = (E // _EB,)
    return pl.pallas_call(
        _edge_proj_body,
        grid=grid,
        in_specs=[
            pl.BlockSpec((_EB, DE), lambda i: (i, 0)),
            pl.BlockSpec((nl, DE, DW), lambda i: (0, 0, 0)),
            pl.BlockSpec((nl, DE, DW), lambda i: (0, 0, 0)),
            pl.BlockSpec((nl, DW), lambda i: (0, 0)),
            pl.BlockSpec((nl, DW), lambda i: (0, 0)),
        ],
        out_specs=[pl.BlockSpec((_EB, DW), lambda i: (i, 0))] * nl,
        out_shape=[jax.ShapeDtypeStruct((E, DW), jnp.int32)] * nl,
    )(edge_attr, We_lo, We_hi, be_lo, be_hi)


def _edge_row_update(mrow, erow, j):
    for g in range(D // 32):
        w = erow[j, pl.ds(16 * g, 16)]
        lo = lax.bitcast_convert_type(w << 16, jnp.float32)
        hi = lax.bitcast_convert_type(w & jnp.int32(-65536), jnp.float32)
        sl_lo = pl.ds(32 * g, 16)
        sl_hi = pl.ds(32 * g + 16, 16)
        mrow[j, sl_lo] = jnp.maximum(mrow[j, sl_lo] + lo, 0.0)
        mrow[j, sl_hi] = jnp.maximum(mrow[j, sl_hi] + hi, 0.0)


@functools.partial(
    pl.kernel,
    out_type=jax.ShapeDtypeStruct((NC, N, D), jnp.float32),
    mesh=plsc.VectorSubcoreMesh(
        core_axis_name="c", subcore_axis_name="s", num_cores=NC, num_subcores=NS
    ),
    scratch_types=[
        [pltpu.VMEM((CH,), jnp.int32)] * 2,
        [pltpu.VMEM((CH,), jnp.int32)] * 2,
        [pltpu.VMEM((CH,), jnp.int32)] * 2,
        [pltpu.VMEM((CH, DW), jnp.int32)] * 2,
        [pltpu.VMEM((CH, D), jnp.float32)] * 2,
        pltpu.VMEM((TAIL,), jnp.int32),
        pltpu.VMEM((TAIL,), jnp.int32),
        pltpu.VMEM((TAIL, DW), jnp.int32),
        pltpu.VMEM((TAIL, D), jnp.float32),
        pltpu.VMEM_SHARED((N, D), jnp.float32),
        [pltpu.SemaphoreType.DMA] * 2,
        [pltpu.SemaphoreType.DMA] * 2,
        [pltpu.SemaphoreType.DMA] * 2,
        [pltpu.SemaphoreType.DMA] * 2,
        pltpu.SemaphoreType.DMA,
    ],
)
def _sc_edge_pass(h_hbm, e_hbm, src_hbm, dst_hbm, out_hbm,
                  sidx, didx, dscat, ebuf, mbuf,
                  tsidx, tdidx, tebuf, tmbuf,
                  aggr, i_sem, e_sem, g_sem, s_sem, t_sem):
    c = lax.axis_index("c")
    s = lax.axis_index("s")
    wid = c * NS + s

    def chunk_base(t):
        return pl.multiple_of(wid * EPW + t * CH, 8)

    def issue_idx_e(t, b):
        base = chunk_base(t)
        pltpu.async_copy(src_hbm.at[pl.ds(base, CH)], sidx[b], i_sem[b])
        pltpu.async_copy(dst_hbm.at[pl.ds(base, CH)], didx[b], i_sem[b])
        pltpu.async_copy(e_hbm.at[pl.ds(base, CH), :], ebuf[b], e_sem[b])

    def wait_idx(t, b):
        base = chunk_base(t)
        pltpu.make_async_copy(
            src_hbm.at[pl.ds(base, CH)], sidx[b], i_sem[b]).wait()
        pltpu.make_async_copy(
            dst_hbm.at[pl.ds(base, CH)], didx[b], i_sem[b]).wait()

    def drain_scatter(b):
        pltpu.make_async_copy(mbuf[b], aggr.at[dscat[b]], s_sem[b]).wait()

    issue_idx_e(0, 0)
    issue_idx_e(1, 1)

    @pl.when(s == 0)
    def _init():
        pltpu.sync_copy(h_hbm, aggr)

    plsc.subcore_barrier()

    wait_idx(0, 0)
    pltpu.async_copy(h_hbm.at[sidx[0]], mbuf[0], g_sem[0])

    def pair_body(i, carry):
        for b in range(2):
            t = 2 * i + b
            base = chunk_base(t)
            nb = 1 - b

            def _launch_next():
                wait_idx(t + 1, nb)
                if b == 0:
                    @pl.when(i >= 1)
                    def _drain_prev():
                        drain_scatter(nb)
                else:
                    drain_scatter(nb)
                pltpu.async_copy(h_hbm.at[sidx[nb]], mbuf[nb], g_sem[nb])

            if b == 0:
                _launch_next()
            else:
                pl.when(i < NPAIR - 1)(_launch_next)

            pltpu.make_async_copy(
                e_hbm.at[pl.ds(base, CH), :], ebuf[b], e_sem[b]).wait()
            pltpu.make_async_copy(
                h_hbm.at[sidx[b]], mbuf[b], g_sem[b]).wait()

            @plsc.parallel_loop(0, CH, 1, unroll=4)
            def _compute(j):
                _edge_row_update(mbuf[b], ebuf[b], j)

            for k in range(CH // 16):
                sl = pl.ds(k * 16, 16)
                dscat[b][sl] = didx[b][sl]
            pltpu.async_copy(mbuf[b], aggr.at[dscat[b]], s_sem[b], add=True)

            @pl.when(i < NPAIR - 1)
            def _prefetch():
                issue_idx_e(t + 2, b)
        return carry

    lax.fori_loop(0, NPAIR, pair_body, 0)

    for b in range(2):
        drain_scatter(b)

    tbase = pl.multiple_of(wid * EPW + NFULL * CH, 8)
    pltpu.sync_copy(src_hbm.at[pl.ds(tbase, TAIL)], tsidx)
    pltpu.sync_copy(dst_hbm.at[pl.ds(tbase, TAIL)], tdidx)
    pltpu.sync_copy(e_hbm.at[pl.ds(tbase, TAIL), :], tebuf)
    pltpu.async_copy(h_hbm.at[tsidx], tmbuf, t_sem).wait()

    def tail_body(j, carry):
        _edge_row_update(tmbuf, tebuf, j)
        return carry

    lax.fori_loop(0, TAIL, tail_body, 0)
    pltpu.sync_copy(tmbuf, aggr.at[tdidx], add=True)

    plsc.subcore_barrier()

    rows = 624

    @pl.when(s < NS - 1)
    def _copy_main():
        off = pl.multiple_of(s * rows, 8)
        pltpu.sync_copy(
            aggr.at[pl.ds(off, rows), :],
            out_hbm.at[c, pl.ds(off, rows), :],
        )

    @pl.when(s == NS - 1)
    def _copy_tail():
        off = (NS - 1) * rows
        pltpu.sync_copy(
            aggr.at[pl.ds(off, N - off), :],
            out_hbm.at[c, pl.ds(off, N - off), :],
        )


_NB = 1000


def _dense_body(agg_ref, h_ref, w1_ref, b1_ref, w2_ref, b2_ref,
                sc_ref, sh_ref, out_ref):
    x = agg_ref[0] + agg_ref[1] - h_ref[...]
    x = jnp.maximum(
        jnp.dot(x, w1_ref[...], preferred_element_type=jnp.float32) + b1_ref[...],
        0.0,
    )
    x = jnp.maximum(
        jnp.dot(x, w2_ref[...], preferred_element_type=jnp.float32) + b2_ref[...],
        0.0,
    )
    x = x * sc_ref[...] + sh_ref[...]
    out_ref[...] = jnp.maximum(x, 0.0)


def _dense(agg, h, w1, b1, w2, b2, scale, shift):
    grid = (N // _NB,)
    return pl.pallas_call(
        _dense_body,
        grid=grid,
        in_specs=[
            pl.BlockSpec((NC, _NB, D), lambda i: (0, i, 0)),
            pl.BlockSpec((_NB, D), lambda i: (i, 0)),
            pl.BlockSpec((D, D), lambda i: (0, 0)),
            pl.BlockSpec((D,), lambda i: (0,)),
            pl.BlockSpec((D, D), lambda i: (0, 0)),
            pl.BlockSpec((D,), lambda i: (0,)),
            pl.BlockSpec((D,), lambda i: (0,)),
            pl.BlockSpec((D,), lambda i: (0,)),
        ],
        out_specs=pl.BlockSpec((_NB, D), lambda i: (i, 0)),
        out_shape=jax.ShapeDtypeStruct((N, D), jnp.float32),
    )(agg, h, w1, b1, w2, b2, scale, shift)


def _pool_body(h_ref, batch_ref, wc1_ref, bc1_ref, wc2_ref, bc2_ref,
               out_ref, sums_ref, cnts_ref):
    i = pl.program_id(0)

    @pl.when(i == 0)
    def _init():
        sums_ref[...] = jnp.zeros_like(sums_ref)
        cnts_ref[...] = jnp.zeros_like(cnts_ref)

    b = batch_ref[...]
    gids = lax.broadcasted_iota(jnp.int32, (_NB, G), 1)
    onehot = (gids == b).astype(jnp.float32)
    dn = (((0,), (0,)), ((), ()))
    sums_ref[...] += lax.dot_general(
        onehot, h_ref[...], dn, preferred_element_type=jnp.float32
    )
    cnts_ref[...] += lax.dot_general(
        onehot, jnp.ones_like(h_ref), dn, preferred_element_type=jnp.float32
    )

    @pl.when(i == (N // _NB) - 1)
    def _finish():
        pooled = sums_ref[...] / jnp.maximum(cnts_ref[...], 1.0)
        hid = jnp.maximum(
            jnp.dot(pooled, wc1_ref[...], preferred_element_type=jnp.float32)
            + bc1_ref[...],
            0.0,
        )
        out_ref[...] = (
            jnp.dot(hid, wc2_ref[...], preferred_element_type=jnp.float32)
            + bc2_ref[...]
        )


def _pool_classify(h, batch2d, wc1, bc1, wc2, bc2):
    grid = (N // _NB,)
    return pl.pallas_call(
        _pool_body,
        grid=grid,
        in_specs=[
            pl.BlockSpec((_NB, D), lambda i: (i, 0)),
            pl.BlockSpec((_NB, 1), lambda i: (i, 0)),
            pl.BlockSpec((D, D), lambda i: (0, 0)),
            pl.BlockSpec((D,), lambda i: (0,)),
            pl.BlockSpec((D, OUT), lambda i: (0, 0)),
            pl.BlockSpec((OUT,), lambda i: (0,)),
        ],
        out_specs=pl.BlockSpec((G, OUT), lambda i: (0, 0)),
        out_shape=jax.ShapeDtypeStruct((G, OUT), jnp.float32),
        scratch_shapes=[
            pltpu.VMEM((G, D), jnp.float32),
            pltpu.VMEM((G, D), jnp.float32),
        ],
    )(h, batch2d, wc1, bc1, wc2, bc2)


def kernel(x, edge_attr, W1, b1, W2, b2, We, be, bn_gamma, bn_beta,
           bn_mean, bn_var, Wc1, bc1, Wc2, bc2, edge_index, batch):
    src = edge_index[0]
    dst = edge_index[1]
    perm = jnp.asarray(_EPERM, dtype=jnp.int32)
    We_p = We[:, :, perm]
    be_p = be[:, perm]
    es = _edge_proj(edge_attr, We_p[:, :, :DW], We_p[:, :, DW:],
                    be_p[:, :DW], be_p[:, DW:])
    scale = bn_gamma / jnp.sqrt(bn_var + BN_EPS)
    shift = bn_beta - bn_mean * scale

    h = x
    for i in range(L):
        agg = _sc_edge_pass(h, es[i], src, dst)
        h = _dense(agg, h, W1[i], b1[i], W2[i], b2[i], scale[i], shift[i])

    return _pool_classify(h, batch.reshape(N, 1), Wc1, bc1, Wc2, bc2)
